# baseline (speedup 1.0000x reference)
import functools

import numpy as np

import jax
import jax.numpy as jnp
from jax.experimental import pallas as pl
from jax.experimental.pallas import tpu as pltpu


_SENT_TILE = 16
_FEAT_BUFS = 4
_OUT_BUFS = 4


def _round_up(x, m):
    return ((x + m - 1) // m) * m


def _pick_perm_tile(p):
    for t in range(min(p, 504), 7, -1):
        if p % t == 0 and t % 8 == 0:
            return t
    return None


def _sum_project(block, w_ref, inv_tokens):
    sums = jnp.sum(block, axis=1) * inv_tokens
    q = jnp.dot(sums, w_ref[...], preferred_element_type=jnp.float32)
    return q.astype(jnp.bfloat16)


def _fused_kernel(feat_hbm, w_hbm, memb1_hbm, memb2_hbm, b_hbm, out_hbm,
                  fbuf, obuf, q_ref, w_ref, memb1_ref, memb2_ref, b_ref,
                  fsem, csem, osem, psem, *,
                  inv_tokens, ts, nsum, tile, npair):
    nfull = nsum - 1
    half = ts // 2
    last_slot = nfull % _FEAT_BUFS

    pltpu.make_async_copy(w_hbm, w_ref, psem.at[0]).start()
    for k in range(min(_FEAT_BUFS, nfull)):
        for h in range(2):
            pltpu.make_async_copy(
                feat_hbm.at[pl.ds(k * ts + h * half, half)],
                fbuf.at[k, pl.ds(h * half, half)], fsem.at[k, h]).start()
    for c in range(2):
        pltpu.make_async_copy(
            feat_hbm.at[pl.ds(nfull * ts + c * half, half)],
            fbuf.at[last_slot, pl.ds(c * half, half)], csem.at[c]).start()
    pltpu.make_async_copy(memb1_hbm, memb1_ref, psem.at[1]).start()
    pltpu.make_async_copy(memb2_hbm, memb2_ref, psem.at[2]).start()
    pltpu.make_async_copy(b_hbm, b_ref, psem.at[3]).start()
    pltpu.make_async_copy(w_ref, w_ref, psem.at[0]).wait()

    for k in range(nfull):
        slot = k % _FEAT_BUFS
        for h in range(2):
            pltpu.make_async_copy(fbuf.at[slot, pl.ds(h * half, half)],
                                  fbuf.at[slot, pl.ds(h * half, half)],
                                  fsem.at[slot, h]).wait()
        q_ref[pl.ds(k * ts, ts), :] = _sum_project(fbuf[slot], w_ref,
                                                   inv_tokens)
        nxt = k + _FEAT_BUFS
        if nxt < nfull:
            for h in range(2):
                pltpu.make_async_copy(
                    feat_hbm.at[pl.ds(nxt * ts + h * half, half)],
                    fbuf.at[slot, pl.ds(h * half, half)],
                    fsem.at[slot, h]).start()

    kpart = nfull * ts
    pltpu.make_async_copy(memb1_ref, memb1_ref, psem.at[1]).wait()
    pltpu.make_async_copy(memb2_ref, memb2_ref, psem.at[2]).wait()
    pltpu.make_async_copy(b_ref, b_ref, psem.at[3]).wait()
    overlap = npair <= _OUT_BUFS and kpart > 0
    if overlap:
        for j in range(npair):
            obuf[j % _OUT_BUFS] = jnp.dot(
                memb1_ref[j * tile:(j + 1) * tile, :],
                q_ref[:kpart, :], preferred_element_type=jnp.float32)

    for c in range(2):
        pltpu.make_async_copy(fbuf.at[last_slot, pl.ds(c * half, half)],
                              fbuf.at[last_slot, pl.ds(c * half, half)],
                              csem.at[c]).wait()
        block = fbuf[last_slot, pl.ds(c * half, half)]
        q_ref[pl.ds(kpart + c * half, half), :] = _sum_project(
            block, w_ref, inv_tokens)

    for j in range(npair):
        oslot = j % _OUT_BUFS
        if not overlap and j >= _OUT_BUFS:
            pltpu.make_async_copy(obuf.at[oslot], obuf.at[oslot],
                                  osem.at[oslot]).wait()
        if overlap:
            pooled = obuf[oslot] + jnp.dot(
                memb2_ref[j * tile:(j + 1) * tile, :],
                q_ref[kpart:, :], preferred_element_type=jnp.float32)
        elif kpart > 0:
            pooled = (jnp.dot(memb1_ref[j * tile:(j + 1) * tile, :],
                              q_ref[:kpart, :],
                              preferred_element_type=jnp.float32)
                      + jnp.dot(memb2_ref[j * tile:(j + 1) * tile, :],
                                q_ref[kpart:, :],
                                preferred_element_type=jnp.float32))
        else:
            pooled = jnp.dot(memb2_ref[j * tile:(j + 1) * tile, :],
                             q_ref[...], preferred_element_type=jnp.float32)
        obuf[oslot] = jnp.tanh(pooled + b_ref[...])
        pltpu.make_async_copy(obuf.at[oslot],
                              out_hbm.at[pl.ds(j * tile, tile)],
                              osem.at[oslot]).start()
    for j in range(max(npair - _OUT_BUFS, 0), npair):
        oslot = j % _OUT_BUFS
        pltpu.make_async_copy(obuf.at[oslot], obuf.at[oslot],
                              osem.at[oslot]).wait()


def _pair_membership(num_sentences, sk, p_pad):
    pairs = [(a, c) for a in range(num_sentences)
             for c in range(a + 1, num_sentences)]
    memb = np.zeros((p_pad, sk), np.float32)
    for i, (a, c) in enumerate(pairs):
        memb[i, a] = 1.0
        memb[i, c] = 1.0
    return len(pairs), memb


def kernel(features, w, b):
    s, nw, d = features.shape
    reasoning_steps = 2
    inv_tokens = 1.0 / float(reasoning_steps * nw)

    s8 = _round_up(s, _SENT_TILE)
    feats = features.astype(jnp.float32)
    if s8 != s:
        feats = jnp.pad(feats, ((0, s8 - s), (0, 0), (0, 0)))
    nsum = s8 // _SENT_TILE

    p = s * (s - 1) // 2
    tile = _pick_perm_tile(p)
    if tile is None:
        tile = 256
        p_pad = _round_up(p, tile)
    else:
        p_pad = p
    npair = p_pad // tile
    _, memb_np = _pair_membership(s, s8, p_pad)
    kpart = (nsum - 1) * _SENT_TILE
    memb1 = jnp.asarray(memb_np[:, :max(kpart, 1)]).astype(jnp.bfloat16)
    memb2 = jnp.asarray(memb_np[:, kpart:]).astype(jnp.bfloat16)

    fused = functools.partial(_fused_kernel, inv_tokens=inv_tokens,
                              ts=_SENT_TILE, nsum=nsum, tile=tile,
                              npair=npair)
    out = pl.pallas_call(
        fused,
        out_shape=jax.ShapeDtypeStruct((p_pad, d), jnp.float32),
        in_specs=[pl.BlockSpec(memory_space=pl.ANY)] * 5,
        out_specs=pl.BlockSpec(memory_space=pl.ANY),
        scratch_shapes=[
            pltpu.VMEM((min(_FEAT_BUFS, nsum), _SENT_TILE, nw, d),
                       jnp.float32),
            pltpu.VMEM((_OUT_BUFS, tile, d), jnp.float32),
            pltpu.VMEM((s8, d), jnp.bfloat16),
            pltpu.VMEM((d, d), jnp.float32),
            pltpu.VMEM(memb1.shape, jnp.bfloat16),
            pltpu.VMEM(memb2.shape, jnp.bfloat16),
            pltpu.VMEM((1, d), jnp.float32),
            pltpu.SemaphoreType.DMA((_FEAT_BUFS, 2)),
            pltpu.SemaphoreType.DMA((2,)),
            pltpu.SemaphoreType.DMA((_OUT_BUFS,)),
            pltpu.SemaphoreType.DMA((4,)),
        ],
    )(feats, w, memb1, memb2, b)
    return out if p_pad == p else out[:p]


# speedup vs baseline: 1.0467x; 1.0467x over previous
import functools

import numpy as np

import jax
import jax.numpy as jnp
from jax.experimental import pallas as pl
from jax.experimental.pallas import tpu as pltpu


_SENT_TILE = 16
_FEAT_BUFS = 4
_OUT_BUFS = 12


def _round_up(x, m):
    return ((x + m - 1) // m) * m


def _pick_perm_tile(p):
    for t in range(min(p, 168), 7, -1):
        if p % t == 0 and t % 8 == 0:
            return t
    return None


def _sum_project(block, w_ref, inv_tokens):
    sums = jnp.sum(block, axis=1) * inv_tokens
    q = jnp.dot(sums, w_ref[...], preferred_element_type=jnp.float32)
    return q.astype(jnp.bfloat16)


def _fused_kernel(feat_hbm, w_hbm, memb1_hbm, memb2_hbm, b_hbm, out_hbm,
                  fbuf, obuf, q_ref, w_ref, memb1_ref, memb2_ref, b_ref,
                  fsem, csem, osem, psem, *,
                  inv_tokens, ts, nsum, tile, npair):
    nfull = nsum - 1
    half = ts // 2
    last_slot = nfull % _FEAT_BUFS

    pltpu.make_async_copy(w_hbm, w_ref, psem.at[0]).start()
    for k in range(min(_FEAT_BUFS, nfull)):
        for h in range(2):
            pltpu.make_async_copy(
                feat_hbm.at[pl.ds(k * ts + h * half, half)],
                fbuf.at[k, pl.ds(h * half, half)], fsem.at[k, h]).start()
    for c in range(2):
        pltpu.make_async_copy(
            feat_hbm.at[pl.ds(nfull * ts + c * half, half)],
            fbuf.at[last_slot, pl.ds(c * half, half)], csem.at[c]).start()
    pltpu.make_async_copy(memb1_hbm, memb1_ref, psem.at[1]).start()
    pltpu.make_async_copy(memb2_hbm, memb2_ref, psem.at[2]).start()
    pltpu.make_async_copy(b_hbm, b_ref, psem.at[3]).start()
    pltpu.make_async_copy(w_ref, w_ref, psem.at[0]).wait()

    for k in range(nfull):
        slot = k % _FEAT_BUFS
        for h in range(2):
            pltpu.make_async_copy(fbuf.at[slot, pl.ds(h * half, half)],
                                  fbuf.at[slot, pl.ds(h * half, half)],
                                  fsem.at[slot, h]).wait()
        q_ref[pl.ds(k * ts, ts), :] = _sum_project(fbuf[slot], w_ref,
                                                   inv_tokens)
        nxt = k + _FEAT_BUFS
        if nxt < nfull:
            for h in range(2):
                pltpu.make_async_copy(
                    feat_hbm.at[pl.ds(nxt * ts + h * half, half)],
                    fbuf.at[slot, pl.ds(h * half, half)],
                    fsem.at[slot, h]).start()

    kpart = nfull * ts
    pltpu.make_async_copy(memb1_ref, memb1_ref, psem.at[1]).wait()
    pltpu.make_async_copy(memb2_ref, memb2_ref, psem.at[2]).wait()
    pltpu.make_async_copy(b_ref, b_ref, psem.at[3]).wait()
    overlap = npair <= _OUT_BUFS and kpart > 0
    if overlap:
        for j in range(npair):
            obuf[j % _OUT_BUFS] = jnp.dot(
                memb1_ref[j * tile:(j + 1) * tile, :],
                q_ref[:kpart, :], preferred_element_type=jnp.float32)

    for c in range(2):
        pltpu.make_async_copy(fbuf.at[last_slot, pl.ds(c * half, half)],
                              fbuf.at[last_slot, pl.ds(c * half, half)],
                              csem.at[c]).wait()
        block = fbuf[last_slot, pl.ds(c * half, half)]
        q_ref[pl.ds(kpart + c * half, half), :] = _sum_project(
            block, w_ref, inv_tokens)

    for j in range(npair):
        oslot = j % _OUT_BUFS
        if not overlap and j >= _OUT_BUFS:
            pltpu.make_async_copy(obuf.at[oslot], obuf.at[oslot],
                                  osem.at[oslot]).wait()
        if overlap:
            pooled = obuf[oslot] + jnp.dot(
                memb2_ref[j * tile:(j + 1) * tile, :],
                q_ref[kpart:, :], preferred_element_type=jnp.float32)
        elif kpart > 0:
            pooled = (jnp.dot(memb1_ref[j * tile:(j + 1) * tile, :],
                              q_ref[:kpart, :],
                              preferred_element_type=jnp.float32)
                      + jnp.dot(memb2_ref[j * tile:(j + 1) * tile, :],
                                q_ref[kpart:, :],
                                preferred_element_type=jnp.float32))
        else:
            pooled = jnp.dot(memb2_ref[j * tile:(j + 1) * tile, :],
                             q_ref[...], preferred_element_type=jnp.float32)
        obuf[oslot] = jnp.tanh(pooled + b_ref[...])
        pltpu.make_async_copy(obuf.at[oslot],
                              out_hbm.at[pl.ds(j * tile, tile)],
                              osem.at[oslot]).start()
    for j in range(max(npair - _OUT_BUFS, 0), npair):
        oslot = j % _OUT_BUFS
        pltpu.make_async_copy(obuf.at[oslot], obuf.at[oslot],
                              osem.at[oslot]).wait()


def _pair_membership(num_sentences, sk, p_pad):
    pairs = [(a, c) for a in range(num_sentences)
             for c in range(a + 1, num_sentences)]
    memb = np.zeros((p_pad, sk), np.float32)
    for i, (a, c) in enumerate(pairs):
        memb[i, a] = 1.0
        memb[i, c] = 1.0
    return len(pairs), memb


def kernel(features, w, b):
    s, nw, d = features.shape
    reasoning_steps = 2
    inv_tokens = 1.0 / float(reasoning_steps * nw)

    s8 = _round_up(s, _SENT_TILE)
    feats = features.astype(jnp.float32)
    if s8 != s:
        feats = jnp.pad(feats, ((0, s8 - s), (0, 0), (0, 0)))
    nsum = s8 // _SENT_TILE

    p = s * (s - 1) // 2
    tile = _pick_perm_tile(p)
    if tile is None:
        tile = 256
        p_pad = _round_up(p, tile)
    else:
        p_pad = p
    npair = p_pad // tile
    _, memb_np = _pair_membership(s, s8, p_pad)
    kpart = (nsum - 1) * _SENT_TILE
    memb1 = jnp.asarray(memb_np[:, :max(kpart, 1)]).astype(jnp.bfloat16)
    memb2 = jnp.asarray(memb_np[:, kpart:]).astype(jnp.bfloat16)

    fused = functools.partial(_fused_kernel, inv_tokens=inv_tokens,
                              ts=_SENT_TILE, nsum=nsum, tile=tile,
                              npair=npair)
    out = pl.pallas_call(
        fused,
        out_shape=jax.ShapeDtypeStruct((p_pad, d), jnp.float32),
        in_specs=[pl.BlockSpec(memory_space=pl.ANY)] * 5,
        out_specs=pl.BlockSpec(memory_space=pl.ANY),
        scratch_shapes=[
            pltpu.VMEM((min(_FEAT_BUFS, nsum), _SENT_TILE, nw, d),
                       jnp.float32),
            pltpu.VMEM((_OUT_BUFS, tile, d), jnp.float32),
            pltpu.VMEM((s8, d), jnp.bfloat16),
            pltpu.VMEM((d, d), jnp.float32),
            pltpu.VMEM(memb1.shape, jnp.bfloat16),
            pltpu.VMEM(memb2.shape, jnp.bfloat16),
            pltpu.VMEM((1, d), jnp.float32),
            pltpu.SemaphoreType.DMA((_FEAT_BUFS, 2)),
            pltpu.SemaphoreType.DMA((2,)),
            pltpu.SemaphoreType.DMA((_OUT_BUFS,)),
            pltpu.SemaphoreType.DMA((4,)),
        ],
    )(feats, w, memb1, memb2, b)
    return out if p_pad == p else out[:p]


# speedup vs baseline: 1.0515x; 1.0046x over previous
import functools

import numpy as np

import jax
import jax.numpy as jnp
from jax.experimental import pallas as pl
from jax.experimental.pallas import tpu as pltpu


_SENT_TILE = 16
_FEAT_BUFS = 4
_OUT_BUFS = 7


def _round_up(x, m):
    return ((x + m - 1) // m) * m


def _pick_perm_tile(p):
    for t in range(min(p, 288), 7, -1):
        if p % t == 0 and t % 8 == 0:
            return t
    return None


def _sum_project(block, w_ref, inv_tokens):
    sums = jnp.sum(block, axis=1) * inv_tokens
    q = jnp.dot(sums, w_ref[...], preferred_element_type=jnp.float32)
    return q.astype(jnp.bfloat16)


def _fused_kernel(feat_hbm, w_hbm, memb1_hbm, memb2_hbm, b_hbm, out_hbm,
                  fbuf, obuf, q_ref, w_ref, memb1_ref, memb2_ref, b_ref,
                  fsem, csem, osem, psem, *,
                  inv_tokens, ts, nsum, tile, npair):
    nfull = nsum - 1
    half = ts // 2
    last_slot = nfull % _FEAT_BUFS

    pltpu.make_async_copy(w_hbm, w_ref, psem.at[0]).start()
    for k in range(min(_FEAT_BUFS, nfull)):
        for h in range(2):
            pltpu.make_async_copy(
                feat_hbm.at[pl.ds(k * ts + h * half, half)],
                fbuf.at[k, pl.ds(h * half, half)], fsem.at[k, h]).start()
    for c in range(2):
        pltpu.make_async_copy(
            feat_hbm.at[pl.ds(nfull * ts + c * half, half)],
            fbuf.at[last_slot, pl.ds(c * half, half)], csem.at[c]).start()
    pltpu.make_async_copy(memb1_hbm, memb1_ref, psem.at[1]).start()
    pltpu.make_async_copy(memb2_hbm, memb2_ref, psem.at[2]).start()
    pltpu.make_async_copy(b_hbm, b_ref, psem.at[3]).start()
    pltpu.make_async_copy(w_ref, w_ref, psem.at[0]).wait()

    for k in range(nfull):
        slot = k % _FEAT_BUFS
        for h in range(2):
            pltpu.make_async_copy(fbuf.at[slot, pl.ds(h * half, half)],
                                  fbuf.at[slot, pl.ds(h * half, half)],
                                  fsem.at[slot, h]).wait()
        q_ref[pl.ds(k * ts, ts), :] = _sum_project(fbuf[slot], w_ref,
                                                   inv_tokens)
        nxt = k + _FEAT_BUFS
        if nxt < nfull:
            for h in range(2):
                pltpu.make_async_copy(
                    feat_hbm.at[pl.ds(nxt * ts + h * half, half)],
                    fbuf.at[slot, pl.ds(h * half, half)],
                    fsem.at[slot, h]).start()

    kpart = nfull * ts
    pltpu.make_async_copy(memb1_ref, memb1_ref, psem.at[1]).wait()
    pltpu.make_async_copy(memb2_ref, memb2_ref, psem.at[2]).wait()
    pltpu.make_async_copy(b_ref, b_ref, psem.at[3]).wait()
    overlap = npair <= _OUT_BUFS and kpart > 0
    if overlap:
        for j in range(npair):
            obuf[j % _OUT_BUFS] = jnp.dot(
                memb1_ref[j * tile:(j + 1) * tile, :],
                q_ref[:kpart, :], preferred_element_type=jnp.float32)

    for c in range(2):
        pltpu.make_async_copy(fbuf.at[last_slot, pl.ds(c * half, half)],
                              fbuf.at[last_slot, pl.ds(c * half, half)],
                              csem.at[c]).wait()
        block = fbuf[last_slot, pl.ds(c * half, half)]
        q_ref[pl.ds(kpart + c * half, half), :] = _sum_project(
            block, w_ref, inv_tokens)

    for j in range(npair):
        oslot = j % _OUT_BUFS
        if not overlap and j >= _OUT_BUFS:
            pltpu.make_async_copy(obuf.at[oslot], obuf.at[oslot],
                                  osem.at[oslot]).wait()
        if overlap:
            pooled = obuf[oslot] + jnp.dot(
                memb2_ref[j * tile:(j + 1) * tile, :],
                q_ref[kpart:, :], preferred_element_type=jnp.float32)
        elif kpart > 0:
            pooled = (jnp.dot(memb1_ref[j * tile:(j + 1) * tile, :],
                              q_ref[:kpart, :],
                              preferred_element_type=jnp.float32)
                      + jnp.dot(memb2_ref[j * tile:(j + 1) * tile, :],
                                q_ref[kpart:, :],
                                preferred_element_type=jnp.float32))
        else:
            pooled = jnp.dot(memb2_ref[j * tile:(j + 1) * tile, :],
                             q_ref[...], preferred_element_type=jnp.float32)
        obuf[oslot] = jnp.tanh(pooled + b_ref[...])
        pltpu.make_async_copy(obuf.at[oslot],
                              out_hbm.at[pl.ds(j * tile, tile)],
                              osem.at[oslot]).start()
    for j in range(max(npair - _OUT_BUFS, 0), npair):
        oslot = j % _OUT_BUFS
        pltpu.make_async_copy(obuf.at[oslot], obuf.at[oslot],
                              osem.at[oslot]).wait()


def _pair_membership(num_sentences, sk, p_pad):
    pairs = [(a, c) for a in range(num_sentences)
             for c in range(a + 1, num_sentences)]
    memb = np.zeros((p_pad, sk), np.float32)
    for i, (a, c) in enumerate(pairs):
        memb[i, a] = 1.0
        memb[i, c] = 1.0
    return len(pairs), memb


def kernel(features, w, b):
    s, nw, d = features.shape
    reasoning_steps = 2
    inv_tokens = 1.0 / float(reasoning_steps * nw)

    s8 = _round_up(s, _SENT_TILE)
    feats = features.astype(jnp.float32)
    if s8 != s:
        feats = jnp.pad(feats, ((0, s8 - s), (0, 0), (0, 0)))
    nsum = s8 // _SENT_TILE

    p = s * (s - 1) // 2
    tile = _pick_perm_tile(p)
    if tile is None:
        tile = 256
        p_pad = _round_up(p, tile)
    else:
        p_pad = p
    npair = p_pad // tile
    _, memb_np = _pair_membership(s, s8, p_pad)
    kpart = (nsum - 1) * _SENT_TILE
    memb1 = jnp.asarray(memb_np[:, :max(kpart, 1)]).astype(jnp.bfloat16)
    memb2 = jnp.asarray(memb_np[:, kpart:]).astype(jnp.bfloat16)

    fused = functools.partial(_fused_kernel, inv_tokens=inv_tokens,
                              ts=_SENT_TILE, nsum=nsum, tile=tile,
                              npair=npair)
    out = pl.pallas_call(
        fused,
        out_shape=jax.ShapeDtypeStruct((p_pad, d), jnp.float32),
        in_specs=[pl.BlockSpec(memory_space=pl.ANY)] * 5,
        out_specs=pl.BlockSpec(memory_space=pl.ANY),
        scratch_shapes=[
            pltpu.VMEM((min(_FEAT_BUFS, nsum), _SENT_TILE, nw, d),
                       jnp.float32),
            pltpu.VMEM((_OUT_BUFS, tile, d), jnp.float32),
            pltpu.VMEM((s8, d), jnp.bfloat16),
            pltpu.VMEM((d, d), jnp.float32),
            pltpu.VMEM(memb1.shape, jnp.bfloat16),
            pltpu.VMEM(memb2.shape, jnp.bfloat16),
            pltpu.VMEM((1, d), jnp.float32),
            pltpu.SemaphoreType.DMA((_FEAT_BUFS, 2)),
            pltpu.SemaphoreType.DMA((2,)),
            pltpu.SemaphoreType.DMA((_OUT_BUFS,)),
            pltpu.SemaphoreType.DMA((4,)),
        ],
    )(feats, w, memb1, memb2, b)
    return out if p_pad == p else out[:p]
